# baseline (speedup 1.0000x reference)
import functools

import jax
import jax.numpy as jnp
from jax import lax
from jax.experimental import pallas as pl
from jax.experimental.pallas import tpu as pltpu
from jax.experimental.pallas import tpu_sc as plsc

N = 10000
E = 320000
NCORE = 2
NSUB = 16
NW = NCORE * NSUB
EPT = E // NW
NCHUNK = 80
CK = EPT // NCHUNK
RPT = N // NSUB
BR = 2000

_MESH = dict(core_axis_name="c", subcore_axis_name="s",
             num_cores=NCORE, num_subcores=NSUB)


@functools.partial(
    pl.kernel,
    out_type=(jax.ShapeDtypeStruct((NCORE, N), jnp.float32),
              jax.ShapeDtypeStruct((NCORE, N), jnp.float32)),
    mesh=plsc.VectorSubcoreMesh(**_MESH),
    scratch_types=[
        pltpu.VMEM((NCHUNK, CK), jnp.int32),
        pltpu.VMEM((NCHUNK, CK), jnp.int32),
        pltpu.VMEM((N,), jnp.float32),
        pltpu.VMEM((CK,), jnp.float32),
        pltpu.SemaphoreType.DMA,
        pltpu.VMEM_SHARED((N,), jnp.float32),
        pltpu.VMEM_SHARED((N,), jnp.float32),
    ],
    compiler_params=pltpu.CompilerParams(needs_layout_passes=False),
)
def _deg_kernel(dst1_hbm, dst2_hbm, zeros_hbm, ones_hbm, out1, out2,
                dstv1, dstv2, buf_v, ones_v, sem, sh1, sh2):
    c = lax.axis_index("c")
    s = lax.axis_index("s")
    wid = c * NSUB + s
    pltpu.sync_copy(zeros_hbm, buf_v)
    pltpu.sync_copy(ones_hbm, ones_v)
    pltpu.sync_copy(dst1_hbm.at[wid], dstv1)
    pltpu.sync_copy(dst2_hbm.at[wid], dstv2)

    @pl.when(s == 0)
    def _():
        pltpu.sync_copy(buf_v, sh1)
        pltpu.sync_copy(buf_v, sh2)

    plsc.subcore_barrier()

    def one_pass(dstv, sh, out):
        def wave(jo, carry):
            for b in range(8):
                pltpu.async_copy(ones_v, sh.at[dstv.at[jo * 8 + b]], sem,
                                 add=True)
            for b in range(8):
                pltpu.make_async_copy(ones_v, sh.at[dstv.at[0]],
                                      sem).wait()
            return carry

        lax.fori_loop(0, NCHUNK // 8, wave, 0)
        plsc.subcore_barrier()

        @pl.when(s == 0)
        def _():
            pltpu.sync_copy(sh, buf_v)
            pltpu.sync_copy(buf_v, out.at[c])

        plsc.subcore_barrier()

    one_pass(dstv1, sh1, out1)
    one_pass(dstv2, sh2, out2)


def _zero_acc(bufrow, acc_s, s):
    def zero_chunk(j, carry):
        k = s + NSUB * j

        @pl.when(k < N // 80)
        def _():
            pltpu.sync_copy(bufrow.at[pl.ds(0, 80)],
                            acc_s.at[pl.ds(k * 80, 80)])

        return carry

    lax.fori_loop(0, pl.cdiv(N // 80, NSUB), zero_chunk, 0)


def _copy_out(acc_s, out, c, s, bufrow):
    def out_chunk(j, carry):
        k = s + NSUB * j

        @pl.when(k < N // 80)
        def _():
            pltpu.sync_copy(acc_s.at[pl.ds(k * 80, 80)],
                            bufrow.at[pl.ds(0, 80)])
            pltpu.sync_copy(bufrow.at[pl.ds(0, 80)],
                            out.at[c, pl.ds(k * 80, 80)])

        return carry

    lax.fori_loop(0, pl.cdiv(N // 80, NSUB), out_chunk, 0)


def _ring_pass(g_hbm, srcv, dstv, bufs, gsem, ssem, acc_s,
               nchunk, nbuf, ahead):
    def gath(j, b):
        pltpu.async_copy(g_hbm.at[srcv.at[j]], bufs[b], gsem[b])

    def scat(j, b):
        pltpu.async_copy(bufs[b], acc_s.at[dstv.at[j]], ssem[b], add=True)

    def wait_g(b):
        pltpu.make_async_copy(g_hbm.at[srcv.at[0]], bufs[b],
                              gsem[b]).wait()

    def wait_s(b):
        pltpu.make_async_copy(bufs[b], acc_s.at[dstv.at[0]],
                              ssem[b]).wait()

    for j in range(ahead):
        gath(j, j % nbuf)
    for j in range(ahead):
        wait_g(j % nbuf)
        scat(j, j % nbuf)
        gath(j + ahead, (j + ahead) % nbuf)

    def body(jo, carry):
        for bb in range(nbuf):
            j = ahead + jo * nbuf + bb

            @pl.when(j < nchunk)
            def _():
                b = (ahead + bb) % nbuf
                wait_g(b)
                scat(j, b)
                wait_s((2 * ahead + bb) % nbuf)

                @pl.when(j + ahead < nchunk)
                def _():
                    gath(j + ahead, (2 * ahead + bb) % nbuf)

        return carry

    lax.fori_loop(0, (nchunk - ahead + nbuf - 1) // nbuf, body, 0)
    for j in range(nchunk - (nbuf - ahead), nchunk):
        wait_s(j % nbuf)


@functools.partial(
    pl.kernel,
    out_type=(jax.ShapeDtypeStruct((NCORE, N, 128), jnp.bfloat16),
              jax.ShapeDtypeStruct((NCORE, N, 64), jnp.bfloat16)),
    mesh=plsc.VectorSubcoreMesh(**_MESH),
    scratch_types=[
        pltpu.VMEM((NCHUNK, CK), jnp.int32),
        pltpu.VMEM((NCHUNK, CK), jnp.int32),
        [pltpu.VMEM((CK, 128), jnp.bfloat16) for _ in range(4)],
        [pltpu.VMEM((CK, 64), jnp.bfloat16) for _ in range(4)],
        [pltpu.SemaphoreType.DMA for _ in range(4)],
        [pltpu.SemaphoreType.DMA for _ in range(4)],
        pltpu.VMEM_SHARED((N, 128), jnp.bfloat16),
        pltpu.VMEM_SHARED((N, 64), jnp.bfloat16),
    ],
    compiler_params=pltpu.CompilerParams(
        needs_layout_passes=False, use_tc_tiling_on_sc=False),
)
def _agg12(g1_hbm, g2_hbm, src1_hbm, dst1_hbm, src2_hbm, dst2_hbm,
           zrow1_hbm, zrow2_hbm, out1, out2, srcv, dstv, bufs1, bufs2,
           gsem, ssem, acc1_s, acc2_s):
    c = lax.axis_index("c")
    s = lax.axis_index("s")
    wid = c * NSUB + s
    pltpu.sync_copy(zrow1_hbm, bufs1[0])
    pltpu.sync_copy(zrow2_hbm, bufs2[0])
    _zero_acc(bufs1[0], acc1_s, s)
    _zero_acc(bufs2[0], acc2_s, s)
    plsc.subcore_barrier()
    pltpu.sync_copy(src1_hbm.at[wid], srcv)
    pltpu.sync_copy(dst1_hbm.at[wid], dstv)
    _ring_pass(g1_hbm, srcv, dstv, bufs1, gsem, ssem, acc1_s,
               NCHUNK, 4, 2)
    pltpu.sync_copy(src2_hbm.at[wid], srcv)
    pltpu.sync_copy(dst2_hbm.at[wid], dstv)
    _ring_pass(g2_hbm, srcv, dstv, bufs2, gsem, ssem, acc2_s,
               NCHUNK, 4, 2)
    plsc.subcore_barrier()
    _copy_out(acc1_s, out1, c, s, bufs1[0])
    _copy_out(acc2_s, out2, c, s, bufs2[0])


@functools.partial(
    pl.kernel,
    out_type=jax.ShapeDtypeStruct((NCORE, N, 64), jnp.bfloat16),
    mesh=plsc.VectorSubcoreMesh(**_MESH),
    scratch_types=[
        pltpu.VMEM((NCHUNK, CK), jnp.int32),
        pltpu.VMEM((NCHUNK, CK), jnp.int32),
        [pltpu.VMEM((CK, 64), jnp.bfloat16) for _ in range(4)],
        [pltpu.SemaphoreType.DMA for _ in range(4)],
        [pltpu.SemaphoreType.DMA for _ in range(4)],
        pltpu.VMEM_SHARED((N, 64), jnp.bfloat16),
    ],
    compiler_params=pltpu.CompilerParams(
        needs_layout_passes=False, use_tc_tiling_on_sc=False),
)
def _agg64(g_hbm, src_hbm, dst_hbm, zrow_hbm, out, srcv, dstv, bufs,
           gsem, ssem, acc_s):
    c = lax.axis_index("c")
    s = lax.axis_index("s")
    wid = c * NSUB + s
    pltpu.sync_copy(zrow_hbm, bufs[0])
    _zero_acc(bufs[0], acc_s, s)
    plsc.subcore_barrier()
    pltpu.sync_copy(src_hbm.at[wid], srcv)
    pltpu.sync_copy(dst_hbm.at[wid], dstv)
    _ring_pass(g_hbm, srcv, dstv, bufs, gsem, ssem, acc_s, NCHUNK, 4, 2)
    plsc.subcore_barrier()
    _copy_out(acc_s, out, c, s, bufs[0])


def _row_spec(d):
    return pl.BlockSpec((BR, d), lambda i: (i, 0))


def _full_spec(r, c):
    return pl.BlockSpec((r, c), lambda i: (0, 0))


def _tc_prep(d1a, d1b, d2a, d2b, x, W20):
    def body(d1a_r, d1b_r, d2a_r, d2b_r, x_r, w_r, g1_r, g2_r):
        dinv1 = lax.rsqrt(d1a_r[...] + d1b_r[...] + 1.0)
        g1_r[...] = (dinv1 * x_r[...]).astype(jnp.bfloat16)
        dinv2 = lax.rsqrt(d2a_r[...] + d2b_r[...] + 1.0)
        g2_r[...] = (dinv2 * jnp.dot(x_r[...], w_r[...],
                                     preferred_element_type=jnp.float32)
                     ).astype(jnp.bfloat16)

    return pl.pallas_call(
        body,
        grid=(N // BR,),
        in_specs=[_row_spec(1)] * 4 + [_row_spec(128), _full_spec(128, 64)],
        out_specs=[_row_spec(128), _row_spec(64)],
        out_shape=[jax.ShapeDtypeStruct((N, 128), jnp.bfloat16),
                   jax.ShapeDtypeStruct((N, 64), jnp.bfloat16)],
    )(d1a, d1b, d2a, d2b, x, W20)


def _tc_mid(a1a, a1b, g1, d1a, d1b, W10, b10, a2a, a2b, g2, d2a, d2b, b20):
    def body(a1a_r, a1b_r, g1_r, d1a_r, d1b_r, w10_r, b10_r,
             a2a_r, a2b_r, g2_r, d2a_r, d2b_r, b20_r, h1_r, g3_r):
        f32 = jnp.float32
        dinv1 = lax.rsqrt(d1a_r[...] + d1b_r[...] + 1.0)
        s1 = dinv1 * (a1a_r[...].astype(f32) + a1b_r[...].astype(f32)
                      + g1_r[...].astype(f32))
        h1_r[...] = jnp.maximum(
            jnp.dot(s1, w10_r[...], preferred_element_type=f32)
            + b10_r[...], 0.0)
        dinv2 = lax.rsqrt(d2a_r[...] + d2b_r[...] + 1.0)
        h2 = dinv2 * (a2a_r[...].astype(f32) + a2b_r[...].astype(f32)
                      + g2_r[...].astype(f32)) + b20_r[...]
        g3_r[...] = (dinv2 * h2).astype(jnp.bfloat16)

    return pl.pallas_call(
        body,
        grid=(N // BR,),
        in_specs=[_row_spec(128)] * 3 + [_row_spec(1)] * 2 +
                 [_full_spec(128, 256), _full_spec(1, 256)] +
                 [_row_spec(64)] * 3 + [_row_spec(1)] * 2 +
                 [_full_spec(1, 64)],
        out_specs=[_row_spec(256), _row_spec(64)],
        out_shape=[jax.ShapeDtypeStruct((N, 256), jnp.float32),
                   jax.ShapeDtypeStruct((N, 64), jnp.bfloat16)],
    )(a1a, a1b, g1, d1a, d1b, W10, b10, a2a, a2b, g2, d2a, d2b, b20)


def _tc_out(a3a, a3b, g3, d2a, d2b, W21, b21, h1, Wfc, bfc):
    def body(a3a_r, a3b_r, g3_r, d2a_r, d2b_r, w21_r, b21_r, h1_r,
             wfc_r, bfc_r, out_r):
        f32 = jnp.float32
        dinv2 = lax.rsqrt(d2a_r[...] + d2b_r[...] + 1.0)
        s3 = dinv2 * (a3a_r[...].astype(f32) + a3b_r[...].astype(f32)
                      + g3_r[...].astype(f32))
        h2p = jnp.maximum(
            jnp.dot(s3, w21_r[...], preferred_element_type=jnp.float32)
            + b21_r[...], 0.0)
        h = h1_r[...] + h2p
        out_r[...] = jnp.dot(h, wfc_r[...],
                             preferred_element_type=jnp.float32) + bfc_r[...]

    return pl.pallas_call(
        body,
        grid=(N // BR,),
        in_specs=[_row_spec(64)] * 3 + [_row_spec(1)] * 2 +
                 [_full_spec(64, 256), _full_spec(1, 256), _row_spec(256),
                  _full_spec(256, 16), _full_spec(1, 16)],
        out_specs=_row_spec(16),
        out_shape=jax.ShapeDtypeStruct((N, 16), jnp.float32),
    )(a3a, a3b, g3, d2a, d2b, W21, b21, h1, Wfc, bfc)


def kernel(x, edge_index1, edge_index2, W10, b10, W20, b20, W21, b21,
           Wfc, bfc):
    src1 = edge_index1[0].reshape(NW, NCHUNK, CK)
    dst1 = edge_index1[1].reshape(NW, NCHUNK, CK)
    src2 = edge_index2[0].reshape(NW, NCHUNK, CK)
    dst2 = edge_index2[1].reshape(NW, NCHUNK, CK)

    zerosN = jnp.zeros((N,), jnp.float32)
    onesCK = jnp.ones((CK,), jnp.float32)
    zrow128 = jnp.zeros((CK, 128), jnp.bfloat16)
    zrow64 = jnp.zeros((CK, 64), jnp.bfloat16)

    deg1p, deg2p = _deg_kernel(dst1, dst2, zerosN, onesCK)
    d1a = deg1p[0].reshape(N, 1)
    d1b = deg1p[1].reshape(N, 1)
    d2a = deg2p[0].reshape(N, 1)
    d2b = deg2p[1].reshape(N, 1)

    g1, g2 = _tc_prep(d1a, d1b, d2a, d2b, x, W20)

    acc1, acc2 = _agg12(g1, g2, src1, dst1, src2, dst2, zrow128, zrow64)

    h1, g3 = _tc_mid(acc1[0], acc1[1], g1, d1a, d1b, W10,
                     b10.reshape(1, -1), acc2[0], acc2[1], g2, d2a, d2b,
                     b20.reshape(1, -1))

    acc3 = _agg64(g3, src2, dst2, zrow64)

    out = _tc_out(acc3[0], acc3[1], g3, d2a, d2b, W21, b21.reshape(1, -1),
                  h1, Wfc, bfc.reshape(1, -1))
    return out


# speedup vs baseline: 1.0473x; 1.0473x over previous
import functools

import jax
import jax.numpy as jnp
from jax import lax
from jax.experimental import pallas as pl
from jax.experimental.pallas import tpu as pltpu
from jax.experimental.pallas import tpu_sc as plsc

N = 10000
E = 320000
NCORE = 2
NSUB = 16
NW = NCORE * NSUB
EPT = E // NW
NCHUNK = 80
CK = EPT // NCHUNK
RPT = N // NSUB
BR = 2000

_MESH = dict(core_axis_name="c", subcore_axis_name="s",
             num_cores=NCORE, num_subcores=NSUB)


@functools.partial(
    pl.kernel,
    out_type=(jax.ShapeDtypeStruct((NCORE, N), jnp.float32),
              jax.ShapeDtypeStruct((NCORE, N), jnp.float32)),
    mesh=plsc.VectorSubcoreMesh(**_MESH),
    scratch_types=[
        pltpu.VMEM((NCHUNK, CK), jnp.int32),
        pltpu.VMEM((NCHUNK, CK), jnp.int32),
        pltpu.VMEM((N,), jnp.float32),
        pltpu.VMEM((CK,), jnp.float32),
        pltpu.SemaphoreType.DMA,
        pltpu.VMEM_SHARED((N,), jnp.float32),
        pltpu.VMEM_SHARED((N,), jnp.float32),
    ],
    compiler_params=pltpu.CompilerParams(needs_layout_passes=False),
)
def _deg_kernel(dst1_hbm, dst2_hbm, zeros_hbm, ones_hbm, out1, out2,
                dstv1, dstv2, buf_v, ones_v, sem, sh1, sh2):
    c = lax.axis_index("c")
    s = lax.axis_index("s")
    wid = c * NSUB + s
    pltpu.sync_copy(zeros_hbm, buf_v)
    pltpu.sync_copy(ones_hbm, ones_v)
    pltpu.sync_copy(dst1_hbm.at[wid], dstv1)
    pltpu.sync_copy(dst2_hbm.at[wid], dstv2)

    @pl.when(s == 0)
    def _():
        pltpu.sync_copy(buf_v, sh1)
        pltpu.sync_copy(buf_v, sh2)

    plsc.subcore_barrier()

    def one_pass(dstv, sh, out):
        def wave(jo, carry):
            for b in range(8):
                pltpu.async_copy(ones_v, sh.at[dstv.at[jo * 8 + b]], sem,
                                 add=True)
            for b in range(8):
                pltpu.make_async_copy(ones_v, sh.at[dstv.at[0]],
                                      sem).wait()
            return carry

        lax.fori_loop(0, NCHUNK // 8, wave, 0)
        plsc.subcore_barrier()

        @pl.when(s == 0)
        def _():
            pltpu.sync_copy(sh, buf_v)
            pltpu.sync_copy(buf_v, out.at[c])

        plsc.subcore_barrier()

    one_pass(dstv1, sh1, out1)
    one_pass(dstv2, sh2, out2)


def _zero_acc(bufrow, acc_s, s):
    def zero_chunk(j, carry):
        k = s + NSUB * j

        @pl.when(k < N // 80)
        def _():
            pltpu.sync_copy(bufrow.at[pl.ds(0, 80)],
                            acc_s.at[pl.ds(k * 80, 80)])

        return carry

    lax.fori_loop(0, pl.cdiv(N // 80, NSUB), zero_chunk, 0)


def _copy_out(acc_s, out, c, s, bufrow):
    def out_chunk(j, carry):
        k = s + NSUB * j

        @pl.when(k < N // 80)
        def _():
            pltpu.sync_copy(acc_s.at[pl.ds(k * 80, 80)],
                            bufrow.at[pl.ds(0, 80)])
            pltpu.sync_copy(bufrow.at[pl.ds(0, 80)],
                            out.at[c, pl.ds(k * 80, 80)])

        return carry

    lax.fori_loop(0, pl.cdiv(N // 80, NSUB), out_chunk, 0)


def _ring_pass(g_hbm, srcv, dstv, bufs, gsem, ssem, acc_s,
               nchunk, nbuf, ahead):
    def gath(j, b):
        pltpu.async_copy(g_hbm.at[srcv.at[j]], bufs[b], gsem[b])

    def scat(j, b):
        pltpu.async_copy(bufs[b], acc_s.at[dstv.at[j]], ssem[b], add=True)

    def wait_g(b):
        pltpu.make_async_copy(g_hbm.at[srcv.at[0]], bufs[b],
                              gsem[b]).wait()

    def wait_s(b):
        pltpu.make_async_copy(bufs[b], acc_s.at[dstv.at[0]],
                              ssem[b]).wait()

    for j in range(ahead):
        gath(j, j % nbuf)
    for j in range(ahead):
        wait_g(j % nbuf)
        scat(j, j % nbuf)
        gath(j + ahead, (j + ahead) % nbuf)

    def body(jo, carry):
        for bb in range(nbuf):
            j = ahead + jo * nbuf + bb

            @pl.when(j < nchunk)
            def _():
                b = (ahead + bb) % nbuf
                wait_g(b)
                scat(j, b)
                wait_s((2 * ahead + bb) % nbuf)

                @pl.when(j + ahead < nchunk)
                def _():
                    gath(j + ahead, (2 * ahead + bb) % nbuf)

        return carry

    lax.fori_loop(0, (nchunk - ahead + nbuf - 1) // nbuf, body, 0)
    for j in range(nchunk - (nbuf - ahead), nchunk):
        wait_s(j % nbuf)


def _make_agg(D, nbuf, ahead):
    @functools.partial(
        pl.kernel,
        out_type=jax.ShapeDtypeStruct((NCORE, N, D), jnp.bfloat16),
        mesh=plsc.VectorSubcoreMesh(**_MESH),
        scratch_types=[
            pltpu.VMEM((NCHUNK, CK), jnp.int32),
            pltpu.VMEM((NCHUNK, CK), jnp.int32),
            [pltpu.VMEM((CK, D), jnp.bfloat16) for _ in range(nbuf)],
            [pltpu.SemaphoreType.DMA for _ in range(nbuf)],
            [pltpu.SemaphoreType.DMA for _ in range(nbuf)],
            pltpu.VMEM_SHARED((N, D), jnp.bfloat16),
        ],
        compiler_params=pltpu.CompilerParams(
            needs_layout_passes=False, use_tc_tiling_on_sc=False),
    )
    def _agg(g_hbm, src_hbm, dst_hbm, zrow_hbm, out, srcv, dstv, bufs,
             gsem, ssem, acc_s):
        c = lax.axis_index("c")
        s = lax.axis_index("s")
        wid = c * NSUB + s
        pltpu.sync_copy(zrow_hbm, bufs[0])
        _zero_acc(bufs[0], acc_s, s)
        plsc.subcore_barrier()
        pltpu.sync_copy(src_hbm.at[wid], srcv)
        pltpu.sync_copy(dst_hbm.at[wid], dstv)
        _ring_pass(g_hbm, srcv, dstv, bufs, gsem, ssem, acc_s,
                   NCHUNK, nbuf, ahead)
        plsc.subcore_barrier()
        _copy_out(acc_s, out, c, s, bufs[0])

    return _agg


_agg128 = _make_agg(128, 4, 2)
_agg64 = _make_agg(64, 4, 2)


def _row_spec(d):
    return pl.BlockSpec((BR, d), lambda i: (i, 0))


def _full_spec(r, c):
    return pl.BlockSpec((r, c), lambda i: (0, 0))


def _tc_prep(d1a, d1b, d2a, d2b, x, W20):
    def body(d1a_r, d1b_r, d2a_r, d2b_r, x_r, w_r, g1_r, g2_r):
        dinv1 = lax.rsqrt(d1a_r[...] + d1b_r[...] + 1.0)
        g1_r[...] = (dinv1 * x_r[...]).astype(jnp.bfloat16)
        dinv2 = lax.rsqrt(d2a_r[...] + d2b_r[...] + 1.0)
        g2_r[...] = (dinv2 * jnp.dot(x_r[...], w_r[...],
                                     preferred_element_type=jnp.float32)
                     ).astype(jnp.bfloat16)

    return pl.pallas_call(
        body,
        grid=(N // BR,),
        in_specs=[_row_spec(1)] * 4 + [_row_spec(128), _full_spec(128, 64)],
        out_specs=[_row_spec(128), _row_spec(64)],
        out_shape=[jax.ShapeDtypeStruct((N, 128), jnp.bfloat16),
                   jax.ShapeDtypeStruct((N, 64), jnp.bfloat16)],
    )(d1a, d1b, d2a, d2b, x, W20)


def _tc_mid(a1a, a1b, g1, d1a, d1b, W10, b10, a2a, a2b, g2, d2a, d2b, b20):
    def body(a1a_r, a1b_r, g1_r, d1a_r, d1b_r, w10_r, b10_r,
             a2a_r, a2b_r, g2_r, d2a_r, d2b_r, b20_r, h1_r, g3_r):
        f32 = jnp.float32
        dinv1 = lax.rsqrt(d1a_r[...] + d1b_r[...] + 1.0)
        s1 = dinv1 * (a1a_r[...].astype(f32) + a1b_r[...].astype(f32)
                      + g1_r[...].astype(f32))
        h1_r[...] = jnp.maximum(
            jnp.dot(s1, w10_r[...], preferred_element_type=f32)
            + b10_r[...], 0.0)
        dinv2 = lax.rsqrt(d2a_r[...] + d2b_r[...] + 1.0)
        h2 = dinv2 * (a2a_r[...].astype(f32) + a2b_r[...].astype(f32)
                      + g2_r[...].astype(f32)) + b20_r[...]
        g3_r[...] = (dinv2 * h2).astype(jnp.bfloat16)

    return pl.pallas_call(
        body,
        grid=(N // BR,),
        in_specs=[_row_spec(128)] * 3 + [_row_spec(1)] * 2 +
                 [_full_spec(128, 256), _full_spec(1, 256)] +
                 [_row_spec(64)] * 3 + [_row_spec(1)] * 2 +
                 [_full_spec(1, 64)],
        out_specs=[_row_spec(256), _row_spec(64)],
        out_shape=[jax.ShapeDtypeStruct((N, 256), jnp.float32),
                   jax.ShapeDtypeStruct((N, 64), jnp.bfloat16)],
    )(a1a, a1b, g1, d1a, d1b, W10, b10, a2a, a2b, g2, d2a, d2b, b20)


def _tc_out(a3a, a3b, g3, d2a, d2b, W21, b21, h1, Wfc, bfc):
    def body(a3a_r, a3b_r, g3_r, d2a_r, d2b_r, w21_r, b21_r, h1_r,
             wfc_r, bfc_r, out_r):
        f32 = jnp.float32
        dinv2 = lax.rsqrt(d2a_r[...] + d2b_r[...] + 1.0)
        s3 = dinv2 * (a3a_r[...].astype(f32) + a3b_r[...].astype(f32)
                      + g3_r[...].astype(f32))
        h2p = jnp.maximum(
            jnp.dot(s3, w21_r[...], preferred_element_type=jnp.float32)
            + b21_r[...], 0.0)
        h = h1_r[...] + h2p
        out_r[...] = jnp.dot(h, wfc_r[...],
                             preferred_element_type=jnp.float32) + bfc_r[...]

    return pl.pallas_call(
        body,
        grid=(N // BR,),
        in_specs=[_row_spec(64)] * 3 + [_row_spec(1)] * 2 +
                 [_full_spec(64, 256), _full_spec(1, 256), _row_spec(256),
                  _full_spec(256, 16), _full_spec(1, 16)],
        out_specs=_row_spec(16),
        out_shape=jax.ShapeDtypeStruct((N, 16), jnp.float32),
    )(a3a, a3b, g3, d2a, d2b, W21, b21, h1, Wfc, bfc)


def kernel(x, edge_index1, edge_index2, W10, b10, W20, b20, W21, b21,
           Wfc, bfc):
    src1 = edge_index1[0].reshape(NW, NCHUNK, CK)
    dst1 = edge_index1[1].reshape(NW, NCHUNK, CK)
    src2 = edge_index2[0].reshape(NW, NCHUNK, CK)
    dst2 = edge_index2[1].reshape(NW, NCHUNK, CK)

    zerosN = jnp.zeros((N,), jnp.float32)
    onesCK = jnp.ones((CK,), jnp.float32)
    zrow128 = jnp.zeros((CK, 128), jnp.bfloat16)
    zrow64 = jnp.zeros((CK, 64), jnp.bfloat16)

    deg1p, deg2p = _deg_kernel(dst1, dst2, zerosN, onesCK)
    d1a = deg1p[0].reshape(N, 1)
    d1b = deg1p[1].reshape(N, 1)
    d2a = deg2p[0].reshape(N, 1)
    d2b = deg2p[1].reshape(N, 1)

    g1, g2 = _tc_prep(d1a, d1b, d2a, d2b, x, W20)

    acc1 = _agg128(g1, src1, dst1, zrow128)
    acc2 = _agg64(g2, src2, dst2, zrow64)

    h1, g3 = _tc_mid(acc1[0], acc1[1], g1, d1a, d1b, W10,
                     b10.reshape(1, -1), acc2[0], acc2[1], g2, d2a, d2b,
                     b20.reshape(1, -1))

    acc3 = _agg64(g3, src2, dst2, zrow64)

    out = _tc_out(acc3[0], acc3[1], g3, d2a, d2b, W21, b21.reshape(1, -1),
                  h1, Wfc, bfc.reshape(1, -1))
    return out


# speedup vs baseline: 1.1194x; 1.0688x over previous
import functools

import jax
import jax.numpy as jnp
from jax import lax
from jax.experimental import pallas as pl
from jax.experimental.pallas import tpu as pltpu
from jax.experimental.pallas import tpu_sc as plsc

N = 10000
E = 320000
NCORE = 2
NSUB = 16
NW = NCORE * NSUB
EPT = E // NW
NCHUNK = 80
CK = EPT // NCHUNK
RPT = N // NSUB
BR = 2000

_MESH = dict(core_axis_name="c", subcore_axis_name="s",
             num_cores=NCORE, num_subcores=NSUB)


@functools.partial(
    pl.kernel,
    out_type=(jax.ShapeDtypeStruct((NCORE, N), jnp.float32),
              jax.ShapeDtypeStruct((NCORE, N), jnp.float32)),
    mesh=plsc.VectorSubcoreMesh(**_MESH),
    scratch_types=[
        pltpu.VMEM((NCHUNK, CK), jnp.int32),
        pltpu.VMEM((NCHUNK, CK), jnp.int32),
        pltpu.VMEM((N,), jnp.float32),
        pltpu.VMEM((CK,), jnp.float32),
        pltpu.SemaphoreType.DMA,
        pltpu.VMEM_SHARED((N,), jnp.float32),
        pltpu.VMEM_SHARED((N,), jnp.float32),
    ],
    compiler_params=pltpu.CompilerParams(needs_layout_passes=False),
)
def _deg_kernel(dst1_hbm, dst2_hbm, zeros_hbm, ones_hbm, out1, out2,
                dstv1, dstv2, buf_v, ones_v, sem, sh1, sh2):
    c = lax.axis_index("c")
    s = lax.axis_index("s")
    wid = c * NSUB + s
    pltpu.sync_copy(zeros_hbm, buf_v)
    pltpu.sync_copy(ones_hbm, ones_v)
    pltpu.sync_copy(dst1_hbm.at[wid], dstv1)
    pltpu.sync_copy(dst2_hbm.at[wid], dstv2)

    @pl.when(s == 0)
    def _():
        pltpu.sync_copy(buf_v, sh1)
        pltpu.sync_copy(buf_v, sh2)

    plsc.subcore_barrier()

    def one_pass(dstv, sh, out):
        def wave(jo, carry):
            for b in range(8):
                pltpu.async_copy(ones_v, sh.at[dstv.at[jo * 8 + b]], sem,
                                 add=True)
            for b in range(8):
                pltpu.make_async_copy(ones_v, sh.at[dstv.at[0]],
                                      sem).wait()
            return carry

        lax.fori_loop(0, NCHUNK // 8, wave, 0)
        plsc.subcore_barrier()

        @pl.when(s == 0)
        def _():
            pltpu.sync_copy(sh, buf_v)
            pltpu.sync_copy(buf_v, out.at[c])

        plsc.subcore_barrier()

    one_pass(dstv1, sh1, out1)
    one_pass(dstv2, sh2, out2)


def _zero_acc(bufrow, acc_s, s):
    def zero_chunk(j, carry):
        k = s + NSUB * j

        @pl.when(k < N // 80)
        def _():
            pltpu.sync_copy(bufrow.at[pl.ds(0, 80)],
                            acc_s.at[pl.ds(k * 80, 80)])

        return carry

    lax.fori_loop(0, pl.cdiv(N // 80, NSUB), zero_chunk, 0)


def _copy_out(acc_s, out, c, s, bufrow):
    def out_chunk(j, carry):
        k = s + NSUB * j

        @pl.when(k < N // 80)
        def _():
            pltpu.sync_copy(acc_s.at[pl.ds(k * 80, 80)],
                            bufrow.at[pl.ds(0, 80)])
            pltpu.sync_copy(bufrow.at[pl.ds(0, 80)],
                            out.at[c, pl.ds(k * 80, 80)])

        return carry

    lax.fori_loop(0, pl.cdiv(N // 80, NSUB), out_chunk, 0)


def _ring_pass(g_hbm, srcv, dstv, bufs, gsem, ssem, acc_s,
               nchunk, nbuf, ahead):
    def gath(j, b):
        pltpu.async_copy(g_hbm.at[srcv.at[j]], bufs[b], gsem[b])

    def scat(j, b):
        pltpu.async_copy(bufs[b], acc_s.at[dstv.at[j]], ssem[b], add=True)

    def wait_g(b):
        pltpu.make_async_copy(g_hbm.at[srcv.at[0]], bufs[b],
                              gsem[b]).wait()

    def wait_s(b):
        pltpu.make_async_copy(bufs[b], acc_s.at[dstv.at[0]],
                              ssem[b]).wait()

    for j in range(ahead):
        gath(j, j % nbuf)
    for j in range(ahead):
        wait_g(j % nbuf)
        scat(j, j % nbuf)
        gath(j + ahead, (j + ahead) % nbuf)

    def body(jo, carry):
        for bb in range(nbuf):
            j = ahead + jo * nbuf + bb

            @pl.when(j < nchunk)
            def _():
                b = (ahead + bb) % nbuf
                wait_g(b)
                scat(j, b)
                wait_s((2 * ahead + bb) % nbuf)

                @pl.when(j + ahead < nchunk)
                def _():
                    gath(j + ahead, (2 * ahead + bb) % nbuf)

        return carry

    lax.fori_loop(0, (nchunk - ahead + nbuf - 1) // nbuf, body, 0)
    for j in range(nchunk - (nbuf - ahead), nchunk):
        wait_s(j % nbuf)


def _make_agg(D, nbuf, ahead):
    @functools.partial(
        pl.kernel,
        out_type=jax.ShapeDtypeStruct((NCORE, N, D), jnp.bfloat16),
        mesh=plsc.VectorSubcoreMesh(**_MESH),
        scratch_types=[
            pltpu.VMEM((NCHUNK, CK), jnp.int32),
            pltpu.VMEM((NCHUNK, CK), jnp.int32),
            [pltpu.VMEM((CK, D), jnp.bfloat16) for _ in range(nbuf)],
            [pltpu.SemaphoreType.DMA for _ in range(nbuf)],
            [pltpu.SemaphoreType.DMA for _ in range(nbuf)],
            pltpu.VMEM_SHARED((N, D), jnp.bfloat16),
        ],
        compiler_params=pltpu.CompilerParams(
            needs_layout_passes=False, use_tc_tiling_on_sc=False),
    )
    def _agg(g_hbm, src_hbm, dst_hbm, zrow_hbm, out, srcv, dstv, bufs,
             gsem, ssem, acc_s):
        c = lax.axis_index("c")
        s = lax.axis_index("s")
        wid = c * NSUB + s
        pltpu.async_copy(src_hbm.at[wid], srcv, gsem[0])
        pltpu.async_copy(dst_hbm.at[wid], dstv, gsem[1])
        pltpu.sync_copy(zrow_hbm, bufs[0])
        _zero_acc(bufs[0], acc_s, s)
        pltpu.make_async_copy(src_hbm.at[wid], srcv, gsem[0]).wait()
        pltpu.make_async_copy(dst_hbm.at[wid], dstv, gsem[1]).wait()
        plsc.subcore_barrier()
        _ring_pass(g_hbm, srcv, dstv, bufs, gsem, ssem, acc_s,
                   NCHUNK, nbuf, ahead)
        plsc.subcore_barrier()
        _copy_out(acc_s, out, c, s, bufs[0])

    return _agg


_agg128 = _make_agg(128, 6, 3)
_agg64 = _make_agg(64, 6, 3)


def _row_spec(d):
    return pl.BlockSpec((BR, d), lambda i: (i, 0))


def _full_spec(r, c):
    return pl.BlockSpec((r, c), lambda i: (0, 0))


def _tc_prep(d1a, d1b, d2a, d2b, x, W20):
    def body(d1a_r, d1b_r, d2a_r, d2b_r, x_r, w_r, g1_r, g2_r):
        dinv1 = lax.rsqrt(d1a_r[...] + d1b_r[...] + 1.0)
        g1_r[...] = (dinv1 * x_r[...]).astype(jnp.bfloat16)
        dinv2 = lax.rsqrt(d2a_r[...] + d2b_r[...] + 1.0)
        g2_r[...] = (dinv2 * jnp.dot(x_r[...], w_r[...],
                                     preferred_element_type=jnp.float32)
                     ).astype(jnp.bfloat16)

    return pl.pallas_call(
        body,
        grid=(N // BR,),
        in_specs=[_row_spec(1)] * 4 + [_row_spec(128), _full_spec(128, 64)],
        out_specs=[_row_spec(128), _row_spec(64)],
        out_shape=[jax.ShapeDtypeStruct((N, 128), jnp.bfloat16),
                   jax.ShapeDtypeStruct((N, 64), jnp.bfloat16)],
    )(d1a, d1b, d2a, d2b, x, W20)


def _tc_mid(a1a, a1b, g1, d1a, d1b, W10, b10, a2a, a2b, g2, d2a, d2b, b20):
    def body(a1a_r, a1b_r, g1_r, d1a_r, d1b_r, w10_r, b10_r,
             a2a_r, a2b_r, g2_r, d2a_r, d2b_r, b20_r, h1_r, g3_r):
        f32 = jnp.float32
        dinv1 = lax.rsqrt(d1a_r[...] + d1b_r[...] + 1.0)
        s1 = dinv1 * (a1a_r[...].astype(f32) + a1b_r[...].astype(f32)
                      + g1_r[...].astype(f32))
        h1_r[...] = jnp.maximum(
            jnp.dot(s1, w10_r[...], preferred_element_type=f32)
            + b10_r[...], 0.0)
        dinv2 = lax.rsqrt(d2a_r[...] + d2b_r[...] + 1.0)
        h2 = dinv2 * (a2a_r[...].astype(f32) + a2b_r[...].astype(f32)
                      + g2_r[...].astype(f32)) + b20_r[...]
        g3_r[...] = (dinv2 * h2).astype(jnp.bfloat16)

    return pl.pallas_call(
        body,
        grid=(N // BR,),
        in_specs=[_row_spec(128)] * 3 + [_row_spec(1)] * 2 +
                 [_full_spec(128, 256), _full_spec(1, 256)] +
                 [_row_spec(64)] * 3 + [_row_spec(1)] * 2 +
                 [_full_spec(1, 64)],
        out_specs=[_row_spec(256), _row_spec(64)],
        out_shape=[jax.ShapeDtypeStruct((N, 256), jnp.float32),
                   jax.ShapeDtypeStruct((N, 64), jnp.bfloat16)],
    )(a1a, a1b, g1, d1a, d1b, W10, b10, a2a, a2b, g2, d2a, d2b, b20)


def _tc_out(a3a, a3b, g3, d2a, d2b, W21, b21, h1, Wfc, bfc):
    def body(a3a_r, a3b_r, g3_r, d2a_r, d2b_r, w21_r, b21_r, h1_r,
             wfc_r, bfc_r, out_r):
        f32 = jnp.float32
        dinv2 = lax.rsqrt(d2a_r[...] + d2b_r[...] + 1.0)
        s3 = dinv2 * (a3a_r[...].astype(f32) + a3b_r[...].astype(f32)
                      + g3_r[...].astype(f32))
        h2p = jnp.maximum(
            jnp.dot(s3, w21_r[...], preferred_element_type=jnp.float32)
            + b21_r[...], 0.0)
        h = h1_r[...] + h2p
        out_r[...] = jnp.dot(h, wfc_r[...],
                             preferred_element_type=jnp.float32) + bfc_r[...]

    return pl.pallas_call(
        body,
        grid=(N // BR,),
        in_specs=[_row_spec(64)] * 3 + [_row_spec(1)] * 2 +
                 [_full_spec(64, 256), _full_spec(1, 256), _row_spec(256),
                  _full_spec(256, 16), _full_spec(1, 16)],
        out_specs=_row_spec(16),
        out_shape=jax.ShapeDtypeStruct((N, 16), jnp.float32),
    )(a3a, a3b, g3, d2a, d2b, W21, b21, h1, Wfc, bfc)


def kernel(x, edge_index1, edge_index2, W10, b10, W20, b20, W21, b21,
           Wfc, bfc):
    src1 = edge_index1[0].reshape(NW, NCHUNK, CK)
    dst1 = edge_index1[1].reshape(NW, NCHUNK, CK)
    src2 = edge_index2[0].reshape(NW, NCHUNK, CK)
    dst2 = edge_index2[1].reshape(NW, NCHUNK, CK)

    zerosN = jnp.zeros((N,), jnp.float32)
    onesCK = jnp.ones((CK,), jnp.float32)
    zrow128 = jnp.zeros((CK, 128), jnp.bfloat16)
    zrow64 = jnp.zeros((CK, 64), jnp.bfloat16)

    deg1p, deg2p = _deg_kernel(dst1, dst2, zerosN, onesCK)
    d1a = deg1p[0].reshape(N, 1)
    d1b = deg1p[1].reshape(N, 1)
    d2a = deg2p[0].reshape(N, 1)
    d2b = deg2p[1].reshape(N, 1)

    g1, g2 = _tc_prep(d1a, d1b, d2a, d2b, x, W20)

    acc1 = _agg128(g1, src1, dst1, zrow128)
    acc2 = _agg64(g2, src2, dst2, zrow64)

    h1, g3 = _tc_mid(acc1[0], acc1[1], g1, d1a, d1b, W10,
                     b10.reshape(1, -1), acc2[0], acc2[1], g2, d2a, d2b,
                     b20.reshape(1, -1))

    acc3 = _agg64(g3, src2, dst2, zrow64)

    out = _tc_out(acc3[0], acc3[1], g3, d2a, d2b, W21, b21.reshape(1, -1),
                  h1, Wfc, bfc.reshape(1, -1))
    return out


# speedup vs baseline: 1.1582x; 1.0347x over previous
import functools

import jax
import jax.numpy as jnp
from jax import lax
from jax.experimental import pallas as pl
from jax.experimental.pallas import tpu as pltpu
from jax.experimental.pallas import tpu_sc as plsc

N = 10000
E = 320000
NCORE = 2
NSUB = 16
NW = NCORE * NSUB
EPT = E // NW
NCHUNK = 80
CK = EPT // NCHUNK
RPT = N // NSUB
BR = 2000

_MESH = dict(core_axis_name="c", subcore_axis_name="s",
             num_cores=NCORE, num_subcores=NSUB)


@functools.partial(
    pl.kernel,
    out_type=(jax.ShapeDtypeStruct((NCORE, N), jnp.float32),
              jax.ShapeDtypeStruct((NCORE, N), jnp.float32)),
    mesh=plsc.VectorSubcoreMesh(**_MESH),
    scratch_types=[
        pltpu.VMEM((NCHUNK, CK), jnp.int32),
        pltpu.VMEM((NCHUNK, CK), jnp.int32),
        pltpu.VMEM((N,), jnp.float32),
        pltpu.VMEM((CK,), jnp.float32),
        [pltpu.SemaphoreType.DMA for _ in range(4)],
        pltpu.VMEM_SHARED((N,), jnp.float32),
        pltpu.VMEM_SHARED((N,), jnp.float32),
    ],
    compiler_params=pltpu.CompilerParams(needs_layout_passes=False),
)
def _deg_kernel(dst1_hbm, dst2_hbm, zeros_hbm, ones_hbm, out1, out2,
                dstv1, dstv2, buf_v, ones_v, sems, sh1, sh2):
    c = lax.axis_index("c")
    s = lax.axis_index("s")
    wid = c * NSUB + s
    pltpu.async_copy(zeros_hbm, buf_v, sems[0])
    pltpu.async_copy(ones_hbm, ones_v, sems[1])
    pltpu.async_copy(dst1_hbm.at[wid], dstv1, sems[2])
    pltpu.async_copy(dst2_hbm.at[wid], dstv2, sems[3])
    pltpu.make_async_copy(zeros_hbm, buf_v, sems[0]).wait()

    @pl.when(s == 0)
    def _():
        pltpu.sync_copy(buf_v, sh1)
        pltpu.sync_copy(buf_v, sh2)

    pltpu.make_async_copy(ones_hbm, ones_v, sems[1]).wait()
    pltpu.make_async_copy(dst1_hbm.at[wid], dstv1, sems[2]).wait()
    pltpu.make_async_copy(dst2_hbm.at[wid], dstv2, sems[3]).wait()
    plsc.subcore_barrier()

    def one_pass(dstv, sh):
        def wave(jo, carry):
            for b in range(16):
                pltpu.async_copy(ones_v, sh.at[dstv.at[jo * 16 + b]],
                                 sems[0], add=True)
            for b in range(16):
                pltpu.make_async_copy(ones_v, sh.at[dstv.at[0]],
                                      sems[0]).wait()
            return carry

        lax.fori_loop(0, NCHUNK // 16, wave, 0)

    one_pass(dstv1, sh1)
    one_pass(dstv2, sh2)
    plsc.subcore_barrier()

    @pl.when(s == 0)
    def _():
        pltpu.sync_copy(sh1, buf_v)
        pltpu.sync_copy(buf_v, out1.at[c])
        pltpu.sync_copy(sh2, buf_v)
        pltpu.sync_copy(buf_v, out2.at[c])


def _zero_acc(bufrow, acc_s, s):
    def zero_chunk(j, carry):
        k = s + NSUB * j

        @pl.when(k < N // 80)
        def _():
            pltpu.sync_copy(bufrow.at[pl.ds(0, 80)],
                            acc_s.at[pl.ds(k * 80, 80)])

        return carry

    lax.fori_loop(0, pl.cdiv(N // 80, NSUB), zero_chunk, 0)


def _copy_out(acc_s, out, c, s, bufrow):
    def out_chunk(j, carry):
        k = s + NSUB * j

        @pl.when(k < N // 80)
        def _():
            pltpu.sync_copy(acc_s.at[pl.ds(k * 80, 80)],
                            bufrow.at[pl.ds(0, 80)])
            pltpu.sync_copy(bufrow.at[pl.ds(0, 80)],
                            out.at[c, pl.ds(k * 80, 80)])

        return carry

    lax.fori_loop(0, pl.cdiv(N // 80, NSUB), out_chunk, 0)


def _ring_pass(g_hbm, srcv, dstv, bufs, gsem, ssem, acc_s,
               nchunk, nbuf, ahead):
    def gath(j, b):
        pltpu.async_copy(g_hbm.at[srcv.at[j]], bufs[b], gsem[b])

    def scat(j, b):
        pltpu.async_copy(bufs[b], acc_s.at[dstv.at[j]], ssem[b], add=True)

    def wait_g(b):
        pltpu.make_async_copy(g_hbm.at[srcv.at[0]], bufs[b],
                              gsem[b]).wait()

    def wait_s(b):
        pltpu.make_async_copy(bufs[b], acc_s.at[dstv.at[0]],
                              ssem[b]).wait()

    for j in range(ahead):
        gath(j, j % nbuf)
    for j in range(ahead):
        wait_g(j % nbuf)
        scat(j, j % nbuf)
        gath(j + ahead, (j + ahead) % nbuf)

    def body(jo, carry):
        for bb in range(nbuf):
            j = ahead + jo * nbuf + bb

            @pl.when(j < nchunk)
            def _():
                b = (ahead + bb) % nbuf
                wait_g(b)
                scat(j, b)
                wait_s((2 * ahead + bb) % nbuf)

                @pl.when(j + ahead < nchunk)
                def _():
                    gath(j + ahead, (2 * ahead + bb) % nbuf)

        return carry

    lax.fori_loop(0, (nchunk - ahead + nbuf - 1) // nbuf, body, 0)
    for j in range(nchunk - (nbuf - ahead), nchunk):
        wait_s(j % nbuf)


def _make_agg(D, nbuf, ahead):
    @functools.partial(
        pl.kernel,
        out_type=jax.ShapeDtypeStruct((NCORE, N, D), jnp.bfloat16),
        mesh=plsc.VectorSubcoreMesh(**_MESH),
        scratch_types=[
            pltpu.VMEM((NCHUNK, CK), jnp.int32),
            pltpu.VMEM((NCHUNK, CK), jnp.int32),
            [pltpu.VMEM((CK, D), jnp.bfloat16) for _ in range(nbuf)],
            [pltpu.SemaphoreType.DMA for _ in range(nbuf)],
            [pltpu.SemaphoreType.DMA for _ in range(nbuf)],
            pltpu.VMEM_SHARED((N, D), jnp.bfloat16),
        ],
        compiler_params=pltpu.CompilerParams(
            needs_layout_passes=False, use_tc_tiling_on_sc=False),
    )
    def _agg(g_hbm, src_hbm, dst_hbm, zrow_hbm, out, srcv, dstv, bufs,
             gsem, ssem, acc_s):
        c = lax.axis_index("c")
        s = lax.axis_index("s")
        wid = c * NSUB + s
        pltpu.async_copy(src_hbm.at[wid], srcv, gsem[0])
        pltpu.async_copy(dst_hbm.at[wid], dstv, gsem[1])
        pltpu.sync_copy(zrow_hbm, bufs[0])
        _zero_acc(bufs[0], acc_s, s)
        pltpu.make_async_copy(src_hbm.at[wid], srcv, gsem[0]).wait()
        pltpu.make_async_copy(dst_hbm.at[wid], dstv, gsem[1]).wait()
        plsc.subcore_barrier()
        _ring_pass(g_hbm, srcv, dstv, bufs, gsem, ssem, acc_s,
                   NCHUNK, nbuf, ahead)
        plsc.subcore_barrier()
        _copy_out(acc_s, out, c, s, bufs[0])

    return _agg


_agg128 = _make_agg(128, 8, 4)
_agg64 = _make_agg(64, 8, 4)


def _row_spec(d):
    return pl.BlockSpec((BR, d), lambda i: (i, 0))


def _full_spec(r, c):
    return pl.BlockSpec((r, c), lambda i: (0, 0))


def _tc_prep(d1a, d1b, d2a, d2b, x, W20):
    def body(d1a_r, d1b_r, d2a_r, d2b_r, x_r, w_r, g1_r, g2_r):
        dinv1 = lax.rsqrt(d1a_r[...] + d1b_r[...] + 1.0)
        g1_r[...] = (dinv1 * x_r[...]).astype(jnp.bfloat16)
        dinv2 = lax.rsqrt(d2a_r[...] + d2b_r[...] + 1.0)
        g2_r[...] = (dinv2 * jnp.dot(x_r[...], w_r[...],
                                     preferred_element_type=jnp.float32)
                     ).astype(jnp.bfloat16)

    return pl.pallas_call(
        body,
        grid=(N // BR,),
        in_specs=[_row_spec(1)] * 4 + [_row_spec(128), _full_spec(128, 64)],
        out_specs=[_row_spec(128), _row_spec(64)],
        out_shape=[jax.ShapeDtypeStruct((N, 128), jnp.bfloat16),
                   jax.ShapeDtypeStruct((N, 64), jnp.bfloat16)],
    )(d1a, d1b, d2a, d2b, x, W20)


def _tc_mid(a1a, a1b, g1, d1a, d1b, W10, b10, a2a, a2b, g2, d2a, d2b, b20):
    def body(a1a_r, a1b_r, g1_r, d1a_r, d1b_r, w10_r, b10_r,
             a2a_r, a2b_r, g2_r, d2a_r, d2b_r, b20_r, h1_r, g3_r):
        f32 = jnp.float32
        dinv1 = lax.rsqrt(d1a_r[...] + d1b_r[...] + 1.0)
        s1 = dinv1 * (a1a_r[...].astype(f32) + a1b_r[...].astype(f32)
                      + g1_r[...].astype(f32))
        h1_r[...] = jnp.maximum(
            jnp.dot(s1, w10_r[...], preferred_element_type=f32)
            + b10_r[...], 0.0)
        dinv2 = lax.rsqrt(d2a_r[...] + d2b_r[...] + 1.0)
        h2 = dinv2 * (a2a_r[...].astype(f32) + a2b_r[...].astype(f32)
                      + g2_r[...].astype(f32)) + b20_r[...]
        g3_r[...] = (dinv2 * h2).astype(jnp.bfloat16)

    return pl.pallas_call(
        body,
        grid=(N // BR,),
        in_specs=[_row_spec(128)] * 3 + [_row_spec(1)] * 2 +
                 [_full_spec(128, 256), _full_spec(1, 256)] +
                 [_row_spec(64)] * 3 + [_row_spec(1)] * 2 +
                 [_full_spec(1, 64)],
        out_specs=[_row_spec(256), _row_spec(64)],
        out_shape=[jax.ShapeDtypeStruct((N, 256), jnp.float32),
                   jax.ShapeDtypeStruct((N, 64), jnp.bfloat16)],
    )(a1a, a1b, g1, d1a, d1b, W10, b10, a2a, a2b, g2, d2a, d2b, b20)


def _tc_out(a3a, a3b, g3, d2a, d2b, W21, b21, h1, Wfc, bfc):
    def body(a3a_r, a3b_r, g3_r, d2a_r, d2b_r, w21_r, b21_r, h1_r,
             wfc_r, bfc_r, out_r):
        f32 = jnp.float32
        dinv2 = lax.rsqrt(d2a_r[...] + d2b_r[...] + 1.0)
        s3 = dinv2 * (a3a_r[...].astype(f32) + a3b_r[...].astype(f32)
                      + g3_r[...].astype(f32))
        h2p = jnp.maximum(
            jnp.dot(s3, w21_r[...], preferred_element_type=jnp.float32)
            + b21_r[...], 0.0)
        h = h1_r[...] + h2p
        out_r[...] = jnp.dot(h, wfc_r[...],
                             preferred_element_type=jnp.float32) + bfc_r[...]

    return pl.pallas_call(
        body,
        grid=(N // BR,),
        in_specs=[_row_spec(64)] * 3 + [_row_spec(1)] * 2 +
                 [_full_spec(64, 256), _full_spec(1, 256), _row_spec(256),
                  _full_spec(256, 16), _full_spec(1, 16)],
        out_specs=_row_spec(16),
        out_shape=jax.ShapeDtypeStruct((N, 16), jnp.float32),
    )(a3a, a3b, g3, d2a, d2b, W21, b21, h1, Wfc, bfc)


def kernel(x, edge_index1, edge_index2, W10, b10, W20, b20, W21, b21,
           Wfc, bfc):
    src1 = edge_index1[0].reshape(NW, NCHUNK, CK)
    dst1 = edge_index1[1].reshape(NW, NCHUNK, CK)
    src2 = edge_index2[0].reshape(NW, NCHUNK, CK)
    dst2 = edge_index2[1].reshape(NW, NCHUNK, CK)

    zerosN = jnp.zeros((N,), jnp.float32)
    onesCK = jnp.ones((CK,), jnp.float32)
    zrow128 = jnp.zeros((CK, 128), jnp.bfloat16)
    zrow64 = jnp.zeros((CK, 64), jnp.bfloat16)

    deg1p, deg2p = _deg_kernel(dst1, dst2, zerosN, onesCK)
    d1a = deg1p[0].reshape(N, 1)
    d1b = deg1p[1].reshape(N, 1)
    d2a = deg2p[0].reshape(N, 1)
    d2b = deg2p[1].reshape(N, 1)

    g1, g2 = _tc_prep(d1a, d1b, d2a, d2b, x, W20)

    acc1 = _agg128(g1, src1, dst1, zrow128)
    acc2 = _agg64(g2, src2, dst2, zrow64)

    h1, g3 = _tc_mid(acc1[0], acc1[1], g1, d1a, d1b, W10,
                     b10.reshape(1, -1), acc2[0], acc2[1], g2, d2a, d2b,
                     b20.reshape(1, -1))

    acc3 = _agg64(g3, src2, dst2, zrow64)

    out = _tc_out(acc3[0], acc3[1], g3, d2a, d2b, W21, b21.reshape(1, -1),
                  h1, Wfc, bfc.reshape(1, -1))
    return out
